# baseline (speedup 1.0000x reference)
import functools

import jax
import jax.numpy as jnp
from jax import lax
from jax.experimental import pallas as pl
from jax.experimental.pallas import tpu as pltpu
from jax.experimental.pallas import tpu_sc as plsc

_GRID = (100, 100, 100)
_D = 32
_NUM_EMB = _GRID[0] * _GRID[1] * _GRID[2]
_N = 65536

_NC = 2
_NS = 16
_NW = _NC * _NS
_PW = _N // _NW
_CH = 128
_NCHUNK = _PW // _CH
_G = _CH // 16

_TB = 128
_DP = 40
_NBLK = _NUM_EMB // _TB
_REM = _NUM_EMB - _NBLK * _TB

_G0 = _GRID[0]
_G01 = _GRID[0] * _GRID[1]
_CORNERS = (
    (0, 0, 0, 0),
    (1, 0, 0, 1),
    (_G0, 0, 1, 0),
    (_G01, 1, 0, 0),
    (_G0 + 1, 0, 1, 1),
    (_G01 + 1, 1, 0, 1),
    (_G01 + _G0, 1, 1, 0),
    (_G01 + _G0 + 1, 1, 1, 1),
)


_SB = 768
_NSB = _NBLK * _TB // _SB
_STEPS = (_NSB + _NW - 1) // _NW


def _relayout_body(tt_hbm, tail_hbm, tlin_hbm, tin0, tin1, tout0, tout1,
                   si0, si1, so0, so1, tailv):
    wid = lax.axis_index("c") * _NS + lax.axis_index("s")
    iotaP = lax.iota(jnp.int32, 16) * _DP
    tins, touts = (tin0, tin1), (tout0, tout1)
    sins, souts = (si0, si1), (so0, so1)

    def sb_of(t):
        return jnp.minimum(t * _NW + wid, _NSB - 1)

    def in_copy(t, s):
        return pltpu.make_async_copy(
            tt_hbm.at[:, pl.ds(sb_of(t) * _SB, _SB)], tins[s], sins[s])

    def out_copy(t, s):
        return pltpu.make_async_copy(
            touts[s], tlin_hbm.at[pl.ds(sb_of(t) * _SB * _DP, _SB * _DP)],
            souts[s])

    def scatter(s):
        def grp(pg, _):
            base = iotaP + pg * (16 * _DP)
            for d in range(_D):
                v = tins[s][d, pl.ds(pg * 16, 16)]
                plsc.store_scatter(touts[s], [base + d], v)
            return _

        lax.fori_loop(0, _SB // 16, grp, None, unroll=4)

    def step(t, s, *, first, refire):
        in_copy(t, s).wait()
        if not first:
            out_copy(t, s).wait()
        scatter(s)
        if refire:
            in_copy(t + 2, s).start()
        out_copy(t, s).start()

    in_copy(0, 0).start()
    in_copy(1, 1).start()
    step(0, 0, first=True, refire=True)
    step(1, 1, first=True, refire=True)

    def pair(i, _):
        t = 2 + i * 2
        step(t, 0, first=False, refire=True)
        step(t + 1, 1, first=False, refire=True)
        return _

    lax.fori_loop(0, (_STEPS - 3) // 2, pair, None, unroll=False)

    step(_STEPS - 1, 0, first=False, refire=False)
    out_copy(_STEPS - 1, 0).wait()
    out_copy(_STEPS - 2, 1).wait()
    in_copy(_STEPS, 1).wait()

    @pl.when(wid == _NW - 1)
    def _rem():
        pltpu.sync_copy(tail_hbm, tailv)
        iota32 = lax.iota(jnp.int32, 16) * _D
        for d in range(_D):
            for pg in range(_REM // 16):
                v = plsc.load_gather(tailv, [iota32 + (pg * 16 * _D + d)])
                plsc.store_scatter(
                    touts[0], [iotaP + (pg * 16 * _DP + d)], v)
        pltpu.sync_copy(
            touts[0].at[pl.ds(0, _REM * _DP)],
            tlin_hbm.at[pl.ds(_NBLK * _TB * _DP, _REM * _DP)])


def _lookup_body(xf_hbm, table_hbm, out_hbm,
                 xv0, xv1, idx0, idx1, wv0, wv1, rows0, rows1,
                 outv0, outv1, sem0, sem1):
    wid = lax.axis_index("c") * _NS + lax.axis_index("s")
    xvs, idxs, wvs = (xv0, xv1), (idx0, idx1), (wv0, wv1)
    rowss, outvs, sems = (rows0, rows1), (outv0, outv1), (sem0, sem1)

    def prep(g, s):
        base = wid * _PW + g * _CH
        pltpu.sync_copy(xf_hbm.at[pl.ds(base * 3, _CH * 3)], xvs[s])

        def grp(i, _):
            sl = pl.ds(i * 16, 16)
            lane3 = lax.iota(jnp.int32, 16) * 3 + i * 48
            rx = plsc.load_gather(xvs[s], [lane3]) * float(_G0)
            ry = plsc.load_gather(xvs[s], [lane3 + 1]) * float(_G0)
            rz = plsc.load_gather(xvs[s], [lane3 + 2]) * float(_G0)
            ix = rx.astype(jnp.int32)
            iy = ry.astype(jnp.int32)
            iz = rz.astype(jnp.int32)
            fx = rx - ix.astype(jnp.float32)
            fy = ry - iy.astype(jnp.float32)
            fz = rz - iz.astype(jnp.float32)
            c000 = ix + iy * _G0 + iz * _G01
            c000 = jnp.where(c000 >= _NUM_EMB, _NUM_EMB - 1, c000)
            ox = (1.0 - fx, fx)
            oy = (1.0 - fy, fy)
            oz = (1.0 - fz, fz)
            for k, (off, a, b, c) in enumerate(_CORNERS):
                ck = c000 + off
                ck = jnp.where(ck >= _NUM_EMB, c000, ck)
                idxs[s][k, sl] = ck
                wvs[s][k, sl] = ox[a] * oy[b] * oz[c]
            return _

        lax.fori_loop(0, _G, grp, None, unroll=False)
        for k in range(8):
            pltpu.make_async_copy(
                table_hbm.at[idxs[s].at[k]], rowss[s].at[k], sems[s]).start()

    def wait_gathers(s):
        for k in range(8):
            pltpu.make_async_copy(
                table_hbm.at[idxs[s].at[k]], rowss[s].at[k], sems[s]).wait()

    def combine(g, s):
        base = wid * _PW + g * _CH
        rows, wv, outv = rowss[s], wvs[s], outvs[s]

        def cmb(i, _):
            sl = pl.ds(i * 16, 16)
            lo = pl.ds(0, 16)
            hi = pl.ds(16, 16)
            wvecs = [wv[k, sl] for k in range(8)]
            for j in range(16):
                pnt = i * 16 + j
                w = wvecs[0][j]
                a0 = rows[0, pnt, lo] * w
                a1 = rows[0, pnt, hi] * w
                for k in range(1, 8):
                    w = wvecs[k][j]
                    a0 = a0 + rows[k, pnt, lo] * w
                    a1 = a1 + rows[k, pnt, hi] * w
                outv[pl.ds(pnt * _D, 16)] = a0
                outv[pl.ds(pnt * _D + 16, 16)] = a1
            return _

        lax.fori_loop(0, _G, cmb, None, unroll=False)
        pltpu.sync_copy(outv, out_hbm.at[pl.ds(base * _D, _CH * _D)])

    prep(0, 0)

    def pair(i, _):
        t = i * 2
        prep(t + 1, 1)
        wait_gathers(0)
        combine(t, 0)
        prep(t + 2, 0)
        wait_gathers(1)
        combine(t + 1, 1)
        return _

    lax.fori_loop(0, _NCHUNK // 2 - 1, pair, None, unroll=False)

    t = _NCHUNK - 2
    prep(t + 1, 1)
    wait_gathers(0)
    combine(t, 0)
    wait_gathers(1)
    combine(t + 1, 1)


@jax.jit
def _grid_embed(x, table):
    mesh = plsc.VectorSubcoreMesh(core_axis_name="c", subcore_axis_name="s")

    relayout = functools.partial(
        pl.kernel,
        mesh=mesh,
        out_type=jax.ShapeDtypeStruct((_NUM_EMB * _DP,), jnp.float32),
        scratch_types=[
            pltpu.VMEM((_D, _SB), jnp.float32),
            pltpu.VMEM((_D, _SB), jnp.float32),
            pltpu.VMEM((_SB * _DP,), jnp.float32),
            pltpu.VMEM((_SB * _DP,), jnp.float32),
            pltpu.SemaphoreType.DMA,
            pltpu.SemaphoreType.DMA,
            pltpu.SemaphoreType.DMA,
            pltpu.SemaphoreType.DMA,
            pltpu.VMEM((_REM * _D,), jnp.float32),
        ],
        compiler_params=pltpu.CompilerParams(
            use_tc_tiling_on_sc=True, needs_layout_passes=False
        ),
    )(_relayout_body)

    lookup = functools.partial(
        pl.kernel,
        mesh=mesh,
        out_type=jax.ShapeDtypeStruct((_N * _D,), jnp.float32),
        scratch_types=[
            pltpu.VMEM((3 * _CH,), jnp.float32),
            pltpu.VMEM((3 * _CH,), jnp.float32),
            pltpu.VMEM((8, _CH), jnp.int32),
            pltpu.VMEM((8, _CH), jnp.int32),
            pltpu.VMEM((8, _CH), jnp.float32),
            pltpu.VMEM((8, _CH), jnp.float32),
            pltpu.VMEM((8, _CH, _DP), jnp.float32),
            pltpu.VMEM((8, _CH, _DP), jnp.float32),
            pltpu.VMEM((_CH * _D,), jnp.float32),
            pltpu.VMEM((_CH * _D,), jnp.float32),
            pltpu.SemaphoreType.DMA,
            pltpu.SemaphoreType.DMA,
        ],
        compiler_params=pltpu.CompilerParams(
            use_tc_tiling_on_sc=False, needs_layout_passes=False
        ),
    )(_lookup_body)

    tail = table[_NBLK * _TB :, :].reshape(-1)
    tlin = relayout(table.T, tail)
    table_lin = tlin.reshape(_NUM_EMB, _DP)
    out = lookup(x.reshape(-1), table_lin)
    return out.reshape(_N, _D)


def kernel(x, table):
    return _grid_embed(x, table)


# speedup vs baseline: 1.0491x; 1.0491x over previous
import functools

import jax
import jax.numpy as jnp
from jax import lax
from jax.experimental import pallas as pl
from jax.experimental.pallas import tpu as pltpu
from jax.experimental.pallas import tpu_sc as plsc

_GRID = (100, 100, 100)
_D = 32
_NUM_EMB = _GRID[0] * _GRID[1] * _GRID[2]
_N = 65536

_NC = 2
_NS = 16
_NW = _NC * _NS
_PW = _N // _NW
_CH = 128
_NCHUNK = _PW // _CH
_G = _CH // 16

_TB = 128
_DP = 40
_NBLK = _NUM_EMB // _TB
_REM = _NUM_EMB - _NBLK * _TB

_G0 = _GRID[0]
_G01 = _GRID[0] * _GRID[1]
_CORNERS = (
    (0, 0, 0, 0),
    (1, 0, 0, 1),
    (_G0, 0, 1, 0),
    (_G01, 1, 0, 0),
    (_G0 + 1, 0, 1, 1),
    (_G01 + 1, 1, 0, 1),
    (_G01 + _G0, 1, 1, 0),
    (_G01 + _G0 + 1, 1, 1, 1),
)


_SB = 768
_NSB = _NBLK * _TB // _SB
_STEPS = (_NSB + _NW - 1) // _NW


def _relayout_body(tt_hbm, tail_hbm, tlin_hbm, tin0, tin1, tout0, tout1,
                   si0, si1, so0, so1, tailv):
    wid = lax.axis_index("c") * _NS + lax.axis_index("s")
    iotaP = lax.iota(jnp.int32, 16) * _DP
    tins, touts = (tin0, tin1), (tout0, tout1)
    sins, souts = (si0, si1), (so0, so1)

    def sb_of(t):
        return jnp.minimum(t * _NW + wid, _NSB - 1)

    def in_copy(t, s):
        return pltpu.make_async_copy(
            tt_hbm.at[:, pl.ds(sb_of(t) * _SB, _SB)], tins[s], sins[s])

    def out_copy(t, s):
        return pltpu.make_async_copy(
            touts[s], tlin_hbm.at[pl.ds(sb_of(t) * _SB * _DP, _SB * _DP)],
            souts[s])

    def scatter(s):
        def grp(pg, _):
            base = iotaP + pg * (16 * _DP)
            for d in range(_D):
                v = tins[s][d, pl.ds(pg * 16, 16)]
                plsc.store_scatter(touts[s], [base + d], v)
            return _

        lax.fori_loop(0, _SB // 16, grp, None, unroll=False)

    def step(t, s, *, first, refire):
        in_copy(t, s).wait()
        if not first:
            out_copy(t, s).wait()
        scatter(s)
        if refire:
            in_copy(t + 2, s).start()
        out_copy(t, s).start()

    in_copy(0, 0).start()
    in_copy(1, 1).start()
    step(0, 0, first=True, refire=True)
    step(1, 1, first=True, refire=True)

    def pair(i, _):
        t = 2 + i * 2
        step(t, 0, first=False, refire=True)
        step(t + 1, 1, first=False, refire=True)
        return _

    lax.fori_loop(0, (_STEPS - 3) // 2, pair, None, unroll=False)

    step(_STEPS - 1, 0, first=False, refire=False)
    out_copy(_STEPS - 1, 0).wait()
    out_copy(_STEPS - 2, 1).wait()
    in_copy(_STEPS, 1).wait()

    @pl.when(wid == _NW - 1)
    def _rem():
        pltpu.sync_copy(tail_hbm, tailv)
        iota32 = lax.iota(jnp.int32, 16) * _D
        for d in range(_D):
            for pg in range(_REM // 16):
                v = plsc.load_gather(tailv, [iota32 + (pg * 16 * _D + d)])
                plsc.store_scatter(
                    touts[0], [iotaP + (pg * 16 * _DP + d)], v)
        pltpu.sync_copy(
            touts[0].at[pl.ds(0, _REM * _DP)],
            tlin_hbm.at[pl.ds(_NBLK * _TB * _DP, _REM * _DP)])


def _lookup_body(xf_hbm, table_hbm, out_hbm,
                 xv0, xv1, idx0, idx1, wv0, wv1, rows0, rows1,
                 outv0, outv1, sem0, sem1):
    wid = lax.axis_index("c") * _NS + lax.axis_index("s")
    xvs, idxs, wvs = (xv0, xv1), (idx0, idx1), (wv0, wv1)
    rowss, outvs, sems = (rows0, rows1), (outv0, outv1), (sem0, sem1)

    def prep(g, s):
        base = wid * _PW + g * _CH
        pltpu.sync_copy(xf_hbm.at[pl.ds(base * 3, _CH * 3)], xvs[s])

        def grp(i, _):
            sl = pl.ds(i * 16, 16)
            lane3 = lax.iota(jnp.int32, 16) * 3 + i * 48
            rx = plsc.load_gather(xvs[s], [lane3]) * float(_G0)
            ry = plsc.load_gather(xvs[s], [lane3 + 1]) * float(_G0)
            rz = plsc.load_gather(xvs[s], [lane3 + 2]) * float(_G0)
            ix = rx.astype(jnp.int32)
            iy = ry.astype(jnp.int32)
            iz = rz.astype(jnp.int32)
            fx = rx - ix.astype(jnp.float32)
            fy = ry - iy.astype(jnp.float32)
            fz = rz - iz.astype(jnp.float32)
            c000 = ix + iy * _G0 + iz * _G01
            c000 = jnp.where(c000 >= _NUM_EMB, _NUM_EMB - 1, c000)
            ox = (1.0 - fx, fx)
            oy = (1.0 - fy, fy)
            oz = (1.0 - fz, fz)
            for k, (off, a, b, c) in enumerate(_CORNERS):
                ck = c000 + off
                ck = jnp.where(ck >= _NUM_EMB, c000, ck)
                idxs[s][k, sl] = ck
                wvs[s][k, sl] = ox[a] * oy[b] * oz[c]
            return _

        lax.fori_loop(0, _G, grp, None, unroll=False)
        for k in range(8):
            pltpu.make_async_copy(
                table_hbm.at[idxs[s].at[k]], rowss[s].at[k], sems[s]).start()

    def wait_gathers(s):
        for k in range(8):
            pltpu.make_async_copy(
                table_hbm.at[idxs[s].at[k]], rowss[s].at[k], sems[s]).wait()

    def combine(g, s):
        base = wid * _PW + g * _CH
        rows, wv, outv = rowss[s], wvs[s], outvs[s]

        def cmb(i, _):
            sl = pl.ds(i * 16, 16)
            lo = pl.ds(0, 16)
            hi = pl.ds(16, 16)
            wvecs = [wv[k, sl] for k in range(8)]
            for j in range(16):
                pnt = i * 16 + j
                w = wvecs[0][j]
                a0 = rows[0, pnt, lo] * w
                a1 = rows[0, pnt, hi] * w
                for k in range(1, 8):
                    w = wvecs[k][j]
                    a0 = a0 + rows[k, pnt, lo] * w
                    a1 = a1 + rows[k, pnt, hi] * w
                outv[pl.ds(pnt * _DP, 16)] = a0
                outv[pl.ds(pnt * _DP + 16, 16)] = a1
            return _

        lax.fori_loop(0, _G, cmb, None, unroll=False)
        pltpu.sync_copy(outv, out_hbm.at[pl.ds(base * _DP, _CH * _DP)])

    prep(0, 0)

    def pair(i, _):
        t = i * 2
        prep(t + 1, 1)
        wait_gathers(0)
        combine(t, 0)
        prep(t + 2, 0)
        wait_gathers(1)
        combine(t + 1, 1)
        return _

    lax.fori_loop(0, _NCHUNK // 2 - 1, pair, None, unroll=False)

    t = _NCHUNK - 2
    prep(t + 1, 1)
    wait_gathers(0)
    combine(t, 0)
    wait_gathers(1)
    combine(t + 1, 1)


_OC = 1024


def _outT_body(of_hbm, outT_hbm, pin, pout):
    wid = lax.axis_index("c") * _NS + lax.axis_index("s")
    iota40 = lax.iota(jnp.int32, 16) * _DP

    for c in range(_PW // _OC):
        base = wid * _PW + c * _OC
        pltpu.sync_copy(of_hbm.at[pl.ds(base * _DP, _OC * _DP)], pin)

        def grp(pg, _):
            idx0 = iota40 + pg * (16 * _DP)
            sl = pl.ds(pg * 16, 16)
            for d in range(_D):
                pout[d, sl] = plsc.load_gather(pin, [idx0 + d])
            return _

        lax.fori_loop(0, _OC // 16, grp, None, unroll=False)
        pltpu.sync_copy(pout, outT_hbm.at[:, pl.ds(base, _OC)])


@jax.jit
def _grid_embed(x, table):
    mesh = plsc.VectorSubcoreMesh(core_axis_name="c", subcore_axis_name="s")

    relayout = functools.partial(
        pl.kernel,
        mesh=mesh,
        out_type=jax.ShapeDtypeStruct((_NUM_EMB * _DP,), jnp.float32),
        scratch_types=[
            pltpu.VMEM((_D, _SB), jnp.float32),
            pltpu.VMEM((_D, _SB), jnp.float32),
            pltpu.VMEM((_SB * _DP,), jnp.float32),
            pltpu.VMEM((_SB * _DP,), jnp.float32),
            pltpu.SemaphoreType.DMA,
            pltpu.SemaphoreType.DMA,
            pltpu.SemaphoreType.DMA,
            pltpu.SemaphoreType.DMA,
            pltpu.VMEM((_REM * _D,), jnp.float32),
        ],
        compiler_params=pltpu.CompilerParams(
            use_tc_tiling_on_sc=True, needs_layout_passes=False
        ),
    )(_relayout_body)

    out_t = functools.partial(
        pl.kernel,
        mesh=mesh,
        out_type=jax.ShapeDtypeStruct((_D, _N), jnp.float32),
        scratch_types=[
            pltpu.VMEM((_OC * _DP,), jnp.float32),
            pltpu.VMEM((_D, _OC), jnp.float32),
        ],
        compiler_params=pltpu.CompilerParams(
            use_tc_tiling_on_sc=True, needs_layout_passes=False
        ),
    )(_outT_body)

    lookup = functools.partial(
        pl.kernel,
        mesh=mesh,
        out_type=jax.ShapeDtypeStruct((_N * _DP,), jnp.float32),
        scratch_types=[
            pltpu.VMEM((3 * _CH,), jnp.float32),
            pltpu.VMEM((3 * _CH,), jnp.float32),
            pltpu.VMEM((8, _CH), jnp.int32),
            pltpu.VMEM((8, _CH), jnp.int32),
            pltpu.VMEM((8, _CH), jnp.float32),
            pltpu.VMEM((8, _CH), jnp.float32),
            pltpu.VMEM((8, _CH, _DP), jnp.float32),
            pltpu.VMEM((8, _CH, _DP), jnp.float32),
            pltpu.VMEM((_CH * _DP,), jnp.float32),
            pltpu.VMEM((_CH * _DP,), jnp.float32),
            pltpu.SemaphoreType.DMA,
            pltpu.SemaphoreType.DMA,
        ],
        compiler_params=pltpu.CompilerParams(
            use_tc_tiling_on_sc=False, needs_layout_passes=False
        ),
    )(_lookup_body)

    tail = table[_NBLK * _TB :, :].reshape(-1)
    tlin = relayout(table.T, tail)
    table_lin = tlin.reshape(_NUM_EMB, _DP)
    out_flat = lookup(x.reshape(-1), table_lin)
    return out_t(out_flat).T


def kernel(x, table):
    return _grid_embed(x, table)
